# baseline (speedup 1.0000x reference)
import functools

import jax
import jax.numpy as jnp
from jax import lax
from jax.experimental import pallas as pl
from jax.experimental.pallas import tpu as pltpu
from jax.experimental.pallas import tpu_sc as plsc

RPAD = 56
AGRP = 8
NBUF = 2
NC = 2
NS = 16
NW = NC * NS


@functools.lru_cache(maxsize=None)
def _build_gather(A, R, V, d):
    a_per_w = A // NW
    ngrp = a_per_w // AGRP
    nloop = ngrp // NBUF
    assert a_per_w * NW == A and ngrp * AGRP == a_per_w and nloop * NBUF == ngrp
    assert R <= RPAD and RPAD % 8 == 0 and RPAD <= 128
    mesh = plsc.VectorSubcoreMesh(core_axis_name="c", subcore_axis_name="s")

    @functools.partial(
        pl.kernel,
        mesh=mesh,
        compiler_params=pltpu.CompilerParams(use_tc_tiling_on_sc=True),
        out_type=jax.ShapeDtypeStruct((A, R, d), jnp.float32),
        scratch_types=[
            pltpu.VMEM((a_per_w * RPAD,), jnp.int32),
            pltpu.VMEM((NBUF, AGRP, RPAD, d), jnp.float32),
        ] + [pltpu.SemaphoreType.DMA] * (2 * NBUF),
    )
    def k(table_hbm, idx_hbm, out_hbm, idx_v, rows_v, *sems):
        sems_g, sems_o = sems[:NBUF], sems[NBUF:]
        wid = lax.axis_index("s") * NC + lax.axis_index("c")
        abase = wid * a_per_w
        pltpu.sync_copy(idx_hbm.at[pl.ds(abase * RPAD, a_per_w * RPAD)], idx_v)

        def gather_desc(g, j, b):
            return pltpu.make_async_copy(
                table_hbm.at[idx_v.at[pl.ds((g * AGRP + j) * RPAD, R)]],
                rows_v.at[b, j, pl.ds(0, R)], sems_g[b])

        def out_desc(g, b):
            return pltpu.make_async_copy(
                rows_v.at[b, :, pl.ds(0, R)],
                out_hbm.at[pl.ds(abase + g * AGRP, AGRP)], sems_o[b])

        def gathers(g, b):
            for j in range(AGRP):
                gather_desc(g, j, b).start()

        def drain(g, b):
            for j in range(AGRP):
                gather_desc(g, j, b).wait()
            out_desc(g, b).start()

        gathers(0, 0)
        gathers(1, 1)
        drain(0, 0)

        def loop(i, carry):
            for b in range(NBUF):
                g = i * NBUF + b
                out_desc(g - NBUF, b).wait()
                gathers(g, b)
                drain(g - 1, (b - 1) % NBUF)
            return carry

        lax.fori_loop(1, nloop, loop, 0)

        drain(ngrp - 1, (ngrp - 1) % NBUF)
        out_desc(ngrp - 2, (ngrp - 2) % NBUF).wait()
        out_desc(ngrp - 1, (ngrp - 1) % NBUF).wait()

    return k


NSPLIT = 2


def kernel(input_tensor, indices):
    d = input_tensor.shape[-1]
    A, R = indices.shape
    idx = jnp.pad(indices.astype(jnp.int32), ((0, 0), (0, RPAD - R)),
                  mode="edge").reshape(-1)
    ah = A // NSPLIT
    gather = _build_gather(ah, R, input_tensor.shape[0], d)
    parts = [gather(input_tensor, idx[k * ah * RPAD:(k + 1) * ah * RPAD])
             for k in range(NSPLIT)]
    return jnp.concatenate(parts, axis=0)


# speedup vs baseline: 1.5958x; 1.5958x over previous
import functools

import jax
import jax.numpy as jnp
from jax import lax
from jax.experimental import pallas as pl
from jax.experimental.pallas import tpu as pltpu
from jax.experimental.pallas import tpu_sc as plsc

RPAD = 56
AGRP = 4
NBUF = 4
NC = 2
NS = 16
NW = NC * NS


@functools.lru_cache(maxsize=None)
def _build_gather(A, R, V, d):
    a_per_w = A // NW
    ngrp = a_per_w // AGRP
    nloop = ngrp // NBUF
    assert a_per_w * NW == A and ngrp * AGRP == a_per_w and nloop * NBUF == ngrp
    assert R <= RPAD and RPAD % 8 == 0 and RPAD <= 128
    mesh = plsc.VectorSubcoreMesh(core_axis_name="c", subcore_axis_name="s")

    @functools.partial(
        pl.kernel,
        mesh=mesh,
        out_type=jax.ShapeDtypeStruct((A, R, d), jnp.float32),
        scratch_types=[
            pltpu.VMEM((a_per_w * RPAD,), jnp.int32),
            pltpu.VMEM((NBUF, AGRP, RPAD, d), jnp.float32),
        ] + [pltpu.SemaphoreType.DMA] * (2 * NBUF),
    )
    def k(table_hbm, idx_hbm, out_hbm, idx_v, rows_v, *sems):
        sems_g, sems_o = sems[:NBUF], sems[NBUF:]
        wid = lax.axis_index("s") * NC + lax.axis_index("c")
        abase = wid * a_per_w
        pltpu.sync_copy(idx_hbm.at[pl.ds(abase * RPAD, a_per_w * RPAD)], idx_v)

        def gather_desc(g, j, b):
            return pltpu.make_async_copy(
                table_hbm.at[idx_v.at[pl.ds((g * AGRP + j) * RPAD, R)]],
                rows_v.at[b, j, pl.ds(0, R)], sems_g[b])

        def out_desc(g, b):
            return pltpu.make_async_copy(
                rows_v.at[b, :, pl.ds(0, R)],
                out_hbm.at[pl.ds(abase + g * AGRP, AGRP)], sems_o[b])

        def gathers(g, b):
            for j in range(AGRP):
                gather_desc(g, j, b).start()

        def drain(g, b):
            for j in range(AGRP):
                gather_desc(g, j, b).wait()
            out_desc(g, b).start()

        for b in range(NBUF):
            gathers(b, b)
        for g in range(NBUF - 1):
            drain(g, g)

        def loop(i, carry):
            for b in range(NBUF):
                g = i * NBUF + b
                out_desc(g - NBUF, b).wait()
                gathers(g, b)
                drain(g - 1, (b - 1) % NBUF)
            return carry

        lax.fori_loop(1, nloop, loop, 0)

        drain(ngrp - 1, (ngrp - 1) % NBUF)
        for g in range(ngrp - NBUF, ngrp):
            out_desc(g, g % NBUF).wait()

    return k


def kernel(input_tensor, indices):
    d = input_tensor.shape[-1]
    A, R = indices.shape
    idx = jnp.pad(indices.astype(jnp.int32), ((0, 0), (0, RPAD - R)),
                  mode="edge")
    return _build_gather(A, R, input_tensor.shape[0], d)(
        input_tensor, idx.reshape(-1))


# speedup vs baseline: 1.6076x; 1.0074x over previous
import functools

import jax
import jax.numpy as jnp
from jax import lax
from jax.experimental import pallas as pl
from jax.experimental.pallas import tpu as pltpu
from jax.experimental.pallas import tpu_sc as plsc

RPAD = 56
AGRP = 8
NBUF = 2
NC = 2
NS = 16
NW = NC * NS


@functools.lru_cache(maxsize=None)
def _build_gather(A, R, V, d):
    a_per_w = A // NW
    ngrp = a_per_w // AGRP
    nloop = ngrp // NBUF
    assert a_per_w * NW == A and ngrp * AGRP == a_per_w and nloop * NBUF == ngrp
    assert R <= RPAD and RPAD % 8 == 0 and RPAD <= 128
    mesh = plsc.VectorSubcoreMesh(core_axis_name="c", subcore_axis_name="s")

    @functools.partial(
        pl.kernel,
        mesh=mesh,
        out_type=jax.ShapeDtypeStruct((A, R, d), jnp.float32),
        scratch_types=[
            pltpu.VMEM((a_per_w * RPAD,), jnp.int32),
            pltpu.VMEM((NBUF, AGRP, RPAD, d), jnp.float32),
        ] + [pltpu.SemaphoreType.DMA] * (2 * NBUF),
    )
    def k(table_hbm, idx_hbm, out_hbm, idx_v, rows_v, *sems):
        sems_g, sems_o = sems[:NBUF], sems[NBUF:]
        wid = lax.axis_index("s") * NC + lax.axis_index("c")
        abase = wid * a_per_w
        pltpu.sync_copy(idx_hbm.at[pl.ds(abase * RPAD, a_per_w * RPAD)], idx_v)

        def gather_desc(g, j, b):
            return pltpu.make_async_copy(
                table_hbm.at[idx_v.at[pl.ds((g * AGRP + j) * RPAD, R)]],
                rows_v.at[b, j, pl.ds(0, R)], sems_g[b])

        def out_desc(g, b):
            return pltpu.make_async_copy(
                rows_v.at[b, :, pl.ds(0, R)],
                out_hbm.at[pl.ds(abase + g * AGRP, AGRP)], sems_o[b])

        def gathers(g, b):
            for j in range(AGRP):
                gather_desc(g, j, b).start()

        def drain(g, b):
            for j in range(AGRP):
                gather_desc(g, j, b).wait()
            out_desc(g, b).start()

        for b in range(NBUF):
            gathers(b, b)
        for g in range(NBUF - 1):
            drain(g, g)

        def loop(i, carry):
            for b in range(NBUF):
                g = i * NBUF + b
                out_desc(g - NBUF, b).wait()
                gathers(g, b)
                drain(g - 1, (b - 1) % NBUF)
            return carry

        lax.fori_loop(1, nloop, loop, 0)

        drain(ngrp - 1, (ngrp - 1) % NBUF)
        for g in range(ngrp - NBUF, ngrp):
            out_desc(g, g % NBUF).wait()

    return k


def kernel(input_tensor, indices):
    d = input_tensor.shape[-1]
    A, R = indices.shape
    idx = jnp.pad(indices.astype(jnp.int32), ((0, 0), (0, RPAD - R)),
                  mode="edge")
    return _build_gather(A, R, input_tensor.shape[0], d)(
        input_tensor, idx.reshape(-1))
